# baseline (speedup 1.0000x reference)
import functools

import jax
import jax.numpy as jnp
import numpy as np
from jax import lax
from jax.experimental import pallas as pl
from jax.experimental.pallas import tpu as pltpu
from jax.experimental.pallas import tpu_sc as plsc

NUM_USER = 50000
NUM_ITEM = 50000
D = 64
B = 1024
E = 800000
L_HIST = 20
S_SIM = 10
L_SIM = 20
S_DIM = 48

N = NUM_USER + 1
NP = 50176
DUMP = N
RT = NP // 16

EB = 320
NBLK = 158
EPAD = 16 * NBLK * EB

DEB = 640
DNBLK = 79


NEG = np.float32(-3.4e38)


def _mesh():
    return plsc.VectorSubcoreMesh(core_axis_name="c", subcore_axis_name="s")


_SC_PARAMS = pltpu.CompilerParams(use_tc_tiling_on_sc=False)


def _sc_degree(dsts2d):
    @functools.partial(
        pl.kernel, mesh=_mesh(),
        out_type=jax.ShapeDtypeStruct((2, NP, 16), jnp.float32),
        scratch_types=[
            pltpu.VMEM((DEB,), jnp.int32),
            pltpu.VMEM((DEB,), jnp.int32),
            pltpu.VMEM((DEB, 16), jnp.float32),
            pltpu.VMEM((DEB, 16), jnp.float32),
            pltpu.VMEM_SHARED((NP, 16), jnp.float32),
            pltpu.SemaphoreType.DMA,
            pltpu.SemaphoreType.DMA,
        ],
        compiler_params=_SC_PARAMS,
    )
    def k(dst_hbm, out_hbm, dbuf0, dbuf1, ones, zbuf, acc, ssem0, ssem1):
        cid = lax.axis_index("c")
        sid = lax.axis_index("s")

        def initrow(i, _):
            ones[i, :] = jnp.full((16,), 1.0, jnp.float32)
            zbuf[i, :] = jnp.full((16,), 0.0, jnp.float32)
            return 0

        lax.fori_loop(0, DEB, initrow, 0)
        r0 = sid * RT
        nz = RT // DEB
        for t in range(nz):
            pltpu.sync_copy(zbuf, acc.at[pl.ds(r0 + t * DEB, DEB)])
        if RT - nz * DEB:
            pltpu.sync_copy(zbuf.at[pl.ds(0, RT - nz * DEB)],
                            acc.at[pl.ds(r0 + nz * DEB, RT - nz * DEB)])
        plsc.subcore_barrier()

        def stage(g, db):
            pltpu.sync_copy(
                dst_hbm.at[cid].at[pl.ds((sid * DNBLK + g) * DEB, DEB)],
                db)

        def fire(db, sem):
            pltpu.async_copy(ones, acc.at[db], sem, add=True)

        def drain(db, sem):
            pltpu.make_async_copy(ones, acc.at[db], sem).wait()

        stage(0, dbuf0)
        fire(dbuf0, ssem0)

        def blk(i2, _):
            g = 2 * i2
            stage(g + 1, dbuf1)
            fire(dbuf1, ssem1)
            drain(dbuf0, ssem0)

            @pl.when(g + 2 < DNBLK)
            def _():
                stage(g + 2, dbuf0)
                fire(dbuf0, ssem0)
            drain(dbuf1, ssem1)
            return 0

        lax.fori_loop(0, DNBLK // 2, blk, 0)
        if DNBLK % 2:
            drain(dbuf0, ssem0)
        plsc.subcore_barrier()
        pltpu.sync_copy(acc.at[pl.ds(r0, RT)],
                        out_hbm.at[cid].at[pl.ds(r0, RT)])

    return k(dsts2d)


def _sc_agg(x2, src2d, dst2d):
    @functools.partial(
        pl.kernel, mesh=_mesh(),
        out_type=jax.ShapeDtypeStruct((2, NP, 32), jnp.float32),
        scratch_types=[
            pltpu.VMEM((EB,), jnp.int32),
            pltpu.VMEM((EB,), jnp.int32),
            pltpu.VMEM((EB,), jnp.int32),
            pltpu.VMEM((EB,), jnp.int32),
            pltpu.VMEM((EB, 32), jnp.float32),
            pltpu.VMEM((EB, 32), jnp.float32),
            pltpu.VMEM_SHARED((NP, 32), jnp.float32),
            pltpu.SemaphoreType.DMA,
            pltpu.SemaphoreType.DMA,
            pltpu.SemaphoreType.DMA,
            pltpu.SemaphoreType.DMA,
        ],
        compiler_params=_SC_PARAMS,
    )
    def k(x2_hbm, src_hbm, dst_hbm, out_hbm, sb0, db0, sb1, db1,
          rows0, rows1, acc, gsem0, gsem1, ssem0, ssem1):
        cid = lax.axis_index("c")
        sid = lax.axis_index("s")

        def zrow(i, _):
            rows0[i, 0:16] = jnp.full((16,), 0.0, jnp.float32)
            rows0[i, 16:32] = jnp.full((16,), 0.0, jnp.float32)
            return 0

        lax.fori_loop(0, EB, zrow, 0)
        r0 = sid * RT
        nz = RT // EB
        for t in range(nz):
            pltpu.sync_copy(rows0, acc.at[pl.ds(r0 + t * EB, EB)])
        if RT - nz * EB:
            pltpu.sync_copy(rows0.at[pl.ds(0, RT - nz * EB)],
                            acc.at[pl.ds(r0 + nz * EB, RT - nz * EB)])
        plsc.subcore_barrier()

        def stage(g, sb, db):
            base = (sid * NBLK + g) * EB
            pltpu.sync_copy(src_hbm.at[pl.ds(base, EB)], sb)
            pltpu.sync_copy(dst_hbm.at[pl.ds(base, EB)], db)

        def fire_g(sb, rows, sem):
            pltpu.async_copy(x2_hbm.at[cid].at[sb], rows, sem)

        def drain_g(sb, rows, sem):
            pltpu.make_async_copy(x2_hbm.at[cid].at[sb], rows, sem).wait()

        def fire_s(rows, db, sem):
            pltpu.async_copy(rows, acc.at[db], sem, add=True)

        def drain_s(rows, db, sem):
            pltpu.make_async_copy(rows, acc.at[db], sem).wait()

        stage(0, sb0, db0)
        fire_g(sb0, rows0, gsem0)

        def blk(i2, _):
            g = 2 * i2

            @pl.when(i2 > 0)
            def _():
                drain_s(rows1, db1, ssem1)
            stage(g + 1, sb1, db1)
            fire_g(sb1, rows1, gsem1)
            drain_g(sb0, rows0, gsem0)
            fire_s(rows0, db0, ssem0)
            drain_s(rows0, db0, ssem0)

            @pl.when(g + 2 < NBLK)
            def _():
                stage(g + 2, sb0, db0)
                fire_g(sb0, rows0, gsem0)
            drain_g(sb1, rows1, gsem1)
            fire_s(rows1, db1, ssem1)
            return 0

        lax.fori_loop(0, NBLK // 2, blk, 0)
        drain_s(rows1, db1, ssem1)
        plsc.subcore_barrier()
        pltpu.sync_copy(acc.at[pl.ds(r0, RT)],
                        out_hbm.at[cid].at[pl.ds(r0, RT)])

    return k(x2, src2d, dst2d)


def _sc_batch(up, it, user, item, iu2d, iiflat, ui2d, uuflat):
    bs = B // 32

    @functools.partial(
        pl.kernel, mesh=_mesh(),
        out_type=[
            jax.ShapeDtypeStruct((B, 64), jnp.float32),
            jax.ShapeDtypeStruct((B, 64), jnp.float32),
            jax.ShapeDtypeStruct((B, 64), jnp.float32),
            jax.ShapeDtypeStruct((B * S_SIM, 64), jnp.float32),
            jax.ShapeDtypeStruct((B * L_HIST, 64), jnp.float32),
            jax.ShapeDtypeStruct((B * S_SIM, 64), jnp.float32),
        ],
        scratch_types=[
            pltpu.VMEM((bs,), jnp.int32),
            pltpu.VMEM((bs,), jnp.int32),
            pltpu.VMEM((bs * L_HIST // 128, 128), jnp.int32),
            pltpu.VMEM((bs * L_SIM * S_SIM,), jnp.int32),
            pltpu.VMEM((bs, 64), jnp.float32),
            pltpu.VMEM((bs, 64), jnp.float32),
            pltpu.VMEM((bs * L_HIST, 64), jnp.float32),
            pltpu.VMEM((L_SIM * S_SIM, 64), jnp.float32),
            pltpu.VMEM((L_SIM * S_SIM, 64), jnp.float32),
            pltpu.VMEM((bs, 64), jnp.float32),
            pltpu.VMEM((bs * S_SIM, 64), jnp.float32),
            pltpu.VMEM((bs * S_SIM, 64), jnp.float32),
            pltpu.SemaphoreType.DMA,
            pltpu.SemaphoreType.DMA,
            pltpu.SemaphoreType.DMA,
        ],
        compiler_params=_SC_PARAMS,
    )
    def k(up_hbm, it_hbm, u_hbm, i_hbm, iu_hbm, ii_hbm, ui_hbm, uu_hbm,
          ue_out, ie_out, iap_out, f_out, xh_out, nf_out,
          ubuf, ibuf, hbuf, sbuf, uerows, ierows, hrows, srows0, srows1,
          iap, fbuf, nfbuf, sem, sm0, sm1):
        cid = lax.axis_index("c")
        sid = lax.axis_index("s")
        wid = sid * 2 + cid
        gb0 = wid * bs
        hch = bs * L_HIST // 128

        pltpu.sync_copy(u_hbm.at[pl.ds(gb0, bs)], ubuf)
        pltpu.sync_copy(i_hbm.at[pl.ds(gb0, bs)], ibuf)
        pltpu.async_copy(up_hbm.at[ubuf], uerows, sem).wait()
        pltpu.async_copy(it_hbm.at[ibuf], ierows, sem).wait()

        pltpu.sync_copy(iu_hbm.at[pl.ds(wid * hch, hch)], hbuf)
        gd = [pltpu.async_copy(up_hbm.at[hbuf.at[j]],
                               hrows.at[pl.ds(j * 128, 128)], sem)
              for j in range(hch)]
        for dsc in gd:
            dsc.wait()

        def iab(b, _):
            for j in range(4):
                ue16 = uerows[b, pl.ds(j * 16, 16)]

                def lb(l, m):
                    r = hrows[b * L_HIST + l, pl.ds(j * 16, 16)]
                    return jnp.maximum(m, r * ue16)

                m = lax.fori_loop(0, L_HIST, lb,
                                  jnp.full((16,), NEG, jnp.float32))
                iap[b, pl.ds(j * 16, 16)] = m
            return 0

        lax.fori_loop(0, bs, iab, 0)

        pltpu.sync_copy(ui_hbm.at[pl.ds(wid * hch, hch)], hbuf)
        gd = [pltpu.async_copy(it_hbm.at[hbuf.at[j]],
                               hrows.at[pl.ds(j * 128, 128)], sem)
              for j in range(hch)]
        for dsc in gd:
            dsc.wait()
        pltpu.sync_copy(hrows, xh_out.at[pl.ds(gb0 * L_HIST, bs * L_HIST)])

        nsim = L_SIM * S_SIM

        def fire_sim(tab, b, rowbuf, sm):
            pltpu.async_copy(tab.at[sbuf.at[pl.ds(b * nsim, nsim)]],
                             rowbuf, sm)

        def drain_sim(tab, b, rowbuf, sm):
            pltpu.make_async_copy(tab.at[sbuf.at[pl.ds(b * nsim, nsim)]],
                                  rowbuf, sm).wait()

        def pool(b, rowbuf, mrows, obuf):
            def sb(s, _2):
                for j in range(4):
                    m16 = mrows[b, pl.ds(j * 16, 16)]

                    def lb(l, m):
                        r = rowbuf[s * L_SIM + l, pl.ds(j * 16, 16)]
                        return jnp.maximum(m, r * m16)

                    m = lax.fori_loop(0, L_SIM, lb,
                                      jnp.full((16,), NEG, jnp.float32))
                    obuf[b * S_SIM + s, pl.ds(j * 16, 16)] = m
                return 0

            lax.fori_loop(0, S_SIM, sb, 0)

        def sim_pass(tab, mrows, obuf):
            fire_sim(tab, 0, srows0, sm0)

            def b2loop(b2, _):
                b = 2 * b2
                fire_sim(tab, b + 1, srows1, sm1)
                drain_sim(tab, b, srows0, sm0)
                pool(b, srows0, mrows, obuf)

                @pl.when(b + 2 < bs)
                def _():
                    fire_sim(tab, b + 2, srows0, sm0)
                drain_sim(tab, b + 1, srows1, sm1)
                pool(b + 1, srows1, mrows, obuf)
                return 0

            lax.fori_loop(0, bs // 2, b2loop, 0)

        pltpu.sync_copy(ii_hbm.at[pl.ds(gb0 * nsim, bs * nsim)], sbuf)
        sim_pass(up_hbm, uerows, fbuf)
        pltpu.sync_copy(uu_hbm.at[pl.ds(gb0 * nsim, bs * nsim)], sbuf)
        sim_pass(it_hbm, ierows, nfbuf)

        pltpu.sync_copy(uerows, ue_out.at[pl.ds(gb0, bs)])
        pltpu.sync_copy(ierows, ie_out.at[pl.ds(gb0, bs)])
        pltpu.sync_copy(iap, iap_out.at[pl.ds(gb0, bs)])
        pltpu.sync_copy(fbuf, f_out.at[pl.ds(gb0 * S_SIM, bs * S_SIM)])
        pltpu.sync_copy(nfbuf, nf_out.at[pl.ds(gb0 * S_SIM, bs * S_SIM)])

    return k(up, it, user, item, iu2d, iiflat, ui2d, uuflat)


def _tc_prep(tab_pad):
    def body(x_ref, o_ref):
        x = x_ref[...]
        o_ref[0] = x[:, :32]
        o_ref[1] = x[:, 32:]

    return pl.pallas_call(
        body,
        grid=(16,),
        in_specs=[pl.BlockSpec((RT, 64), lambda i: (i, 0))],
        out_specs=pl.BlockSpec((2, RT, 32), lambda i: (0, i, 0)),
        out_shape=jax.ShapeDtypeStruct((2, NP, 32), jnp.float32),
    )(tab_pad)


def _tc_norm(raw, deg16, prev, final):
    def body(raw_ref, deg_ref, prev_ref, *out_refs):
        i = pl.program_id(0)
        raw = raw_ref[...]
        d = jnp.maximum(deg_ref[:, 0:1], 1.0)
        a0 = raw[0] / d
        a1 = raw[1] / d
        nsq = (jnp.sum(a0 * a0, axis=1, keepdims=True)
               + jnp.sum(a1 * a1, axis=1, keepdims=True))
        inv = 1.0 / jnp.maximum(jnp.sqrt(nsq), 1e-12)
        n = jnp.concatenate([a0 * inv, a1 * inv], axis=1)
        if final:
            gi = i * RT + lax.broadcasted_iota(jnp.int32, (RT, 1), 0)
            out = (prev_ref[...] + n) * jnp.float32(1.0 / 3.0)
            out_refs[0][...] = jnp.where(gi == 0, 0.0, out)
        else:
            out_refs[0][0] = a0
            out_refs[0][1] = a1
            out_refs[1][...] = prev_ref[...] + n

    if final:
        out_shape = [jax.ShapeDtypeStruct((NP, 64), jnp.float32)]
        out_specs = [pl.BlockSpec((RT, 64), lambda i: (i, 0))]
    else:
        out_shape = [jax.ShapeDtypeStruct((2, NP, 32), jnp.float32),
                     jax.ShapeDtypeStruct((NP, 64), jnp.float32)]
        out_specs = [pl.BlockSpec((2, RT, 32), lambda i: (0, i, 0)),
                     pl.BlockSpec((RT, 64), lambda i: (i, 0))]
    res = pl.pallas_call(
        body,
        grid=(16,),
        in_specs=[pl.BlockSpec((2, RT, 32), lambda i: (0, i, 0)),
                  pl.BlockSpec((RT, 16), lambda i: (i, 0)),
                  pl.BlockSpec((RT, 64), lambda i: (i, 0))],
        out_specs=out_specs,
        out_shape=out_shape,
    )(raw, deg16, prev)
    return res[0] if final else res


def _tc_dense(ue, ie, iapre, f, xh, nf, lens_ii, lens_uu, w):
    BB = 128
    SQD = np.float32(1.0 / np.sqrt(D))
    SQH = np.float32(1.0 / np.sqrt(32))

    def body(ue_ref, ie_ref, iap_ref, f_ref, xh_ref, nf_ref, li_ref, lu_ref,
             wii_ref, wuu_ref, wq1_ref, wk1_ref, wv1_ref, wo1_ref,
             wq2_ref, wk2_ref, wv2_ref, wo2_ref, wf1_ref, bf1_ref,
             wf2_ref, bf2_ref, w1s_ref, b1s_ref, w2s_ref, b2s_ref, out_ref):
        uev = ue_ref[...]
        iev = ie_ref[...]
        iap = iap_ref[...]
        fv = f_ref[...]
        nfv = nf_ref[...]
        xhv = xh_ref[...]
        li = li_ref[...].reshape(BB)
        lu = lu_ref[...].reshape(BB)

        def sim_fuse(active, fe, lens, wmat):
            act = active @ wmat
            s = jnp.sum(act[:, None, :] * fe, axis=-1) * SQD
            mask = (lax.broadcasted_iota(jnp.int32, (BB, S_SIM), 1)
                    < lens[:, None])
            s = jnp.where(mask, s, -1e9)
            a = jax.nn.softmax(s, axis=-1)
            return jnp.sum(a[:, :, None] * fe, axis=1)

        item_neigh = sim_fuse(iap, fv, li, wii_ref[...])
        ia = 0.5 * (iap + item_neigh)

        xf = xhv.reshape(BB * L_HIST, D)
        q = (xf @ wq1_ref[...]).reshape(BB, L_HIST, D)
        kk = (xf @ wk1_ref[...]).reshape(BB, L_HIST, D)
        vv = (xf @ wv1_ref[...]).reshape(BB, L_HIST, D)
        outs = []
        for h in range(2):
            qh = q[:, :, h * 32:(h + 1) * 32]
            kh = kk[:, :, h * 32:(h + 1) * 32]
            vh = vv[:, :, h * 32:(h + 1) * 32]
            sh = lax.dot_general(qh, kh, (((2,), (2,)), ((0,), (0,))),
                                 preferred_element_type=jnp.float32) * SQH
            ah = jax.nn.softmax(sh, axis=-1)
            oh = lax.dot_general(ah, vh, (((2,), (1,)), ((0,), (0,))),
                                 preferred_element_type=jnp.float32)
            outs.append(oh)
        o = jnp.concatenate(outs, axis=-1).reshape(BB * L_HIST, D)
        x_ = o @ wo1_ref[...]
        h1 = xf + x_
        x = (h1 + jnp.maximum(h1 @ wf1_ref[...] + bf1_ref[...], 0.0)
             @ wf2_ref[...] + bf2_ref[...])
        x3 = x.reshape(BB, L_HIST, D)

        q2 = iev @ wq2_ref[...]
        k2 = (x @ wk2_ref[...]).reshape(BB, L_HIST, D)
        v2 = (x @ wv2_ref[...]).reshape(BB, L_HIST, D)
        s2 = jnp.sum(q2[:, None, :] * k2, axis=-1) * SQD
        a2 = jax.nn.softmax(s2, axis=-1)
        o2 = jnp.sum(a2[:, :, None] * v2, axis=1)
        uiv = o2 @ wo2_ref[...]

        user_neigh = sim_fuse(uiv, nfv, lu, wuu_ref[...])
        ui = 0.5 * (uiv + user_neigh)

        lefts = (uev, ui, uev, ui)
        rights = (iev, iev, ia, ia)
        cols = []
        for kq in range(4):
            hh = (lefts[kq] @ w1s_ref[kq, 0] + rights[kq] @ w1s_ref[kq, 1]
                  + b1s_ref[kq][None, :])
            hh = jnp.maximum(hh, 0.0)
            sc = jnp.sum(hh * w2s_ref[kq, :S_DIM][None, :], axis=1,
                         keepdims=True)
            cols.append(sc)
        out_ref[...] = jnp.concatenate(cols, axis=1) + b2s_ref[...]

    nb = B // BB
    full = lambda shape: pl.BlockSpec(shape, lambda i: tuple(0 for _ in shape))
    in_specs = [
        pl.BlockSpec((BB, 64), lambda i: (i, 0)),
        pl.BlockSpec((BB, 64), lambda i: (i, 0)),
        pl.BlockSpec((BB, 64), lambda i: (i, 0)),
        pl.BlockSpec((BB, S_SIM, 64), lambda i: (i, 0, 0)),
        pl.BlockSpec((BB, L_HIST, 64), lambda i: (i, 0, 0)),
        pl.BlockSpec((BB, S_SIM, 64), lambda i: (i, 0, 0)),
        pl.BlockSpec((1, 1, BB), lambda i: (i, 0, 0)),
        pl.BlockSpec((1, 1, BB), lambda i: (i, 0, 0)),
        full((D, D)), full((D, D)),
        full((D, D)), full((D, D)), full((D, D)), full((D, D)),
        full((D, D)), full((D, D)), full((D, D)), full((D, D)),
        full((D, D)), full((1, D)), full((D, D)), full((1, D)),
        full((4, 2, D, S_DIM)), full((4, S_DIM)), full((4, D)), full((1, 4)),
    ]
    return pl.pallas_call(
        body,
        grid=(nb,),
        in_specs=in_specs,
        out_specs=pl.BlockSpec((BB, 4), lambda i: (i, 0)),
        out_shape=jax.ShapeDtypeStruct((B, 4), jnp.float32),
    )(ue, ie, iapre, f.reshape(B, S_SIM, 64), xh.reshape(B, L_HIST, 64),
      nf.reshape(B, S_SIM, 64),
      lens_ii.reshape(nb, 1, BB), lens_uu.reshape(nb, 1, BB), *w)


def _prep_edges(ei):
    pad = EPAD - E
    src = jnp.concatenate([ei[0].astype(jnp.int32),
                           jnp.full((pad,), DUMP, jnp.int32)])
    dst = jnp.concatenate([ei[1].astype(jnp.int32),
                           jnp.full((pad,), DUMP, jnp.int32)])
    return src, dst


def _graph_tables(table, src2d, dst2d, deg16):
    tab_pad = jnp.pad(table, ((0, NP - N), (0, 0)))
    x2 = _tc_prep(tab_pad)
    raw1 = _sc_agg(x2, src2d, dst2d)
    a1, acc1 = _tc_norm(raw1, deg16, tab_pad, final=False)
    raw2 = _sc_agg(a1, src2d, dst2d)
    return _tc_norm(raw2, deg16, acc1, final=True)


def kernel(user, item, user_edge_index, item_edge_index, item_users,
           ii_sim_users, ii_sim_lens, user_items, uu_sim_items, uu_sim_lens,
           params):
    p = params
    usrc, udst = _prep_edges(user_edge_index)
    isrc, idst = _prep_edges(item_edge_index)

    deg2 = _sc_degree(jnp.stack([udst, idst]))
    up_fin = _graph_tables(p['user_table'], usrc, udst, deg2[0])
    it_fin = _graph_tables(p['item_table'], isrc, idst, deg2[1])

    iu2d = item_users.astype(jnp.int32).reshape(-1, 128)
    ui2d = user_items.astype(jnp.int32).reshape(-1, 128)
    iiflat = ii_sim_users.astype(jnp.int32).reshape(-1)
    uuflat = uu_sim_items.astype(jnp.int32).reshape(-1)

    ue, ie, iapre, f, xh, nf = _sc_batch(
        up_fin, it_fin, user.astype(jnp.int32), item.astype(jnp.int32),
        iu2d, iiflat, ui2d, uuflat)

    w = (p['W_ii'], p['W_uu'],
         p['Wq1'], p['Wk1'], p['Wv1'], p['Wo1'],
         p['Wq2'], p['Wk2'], p['Wv2'], p['Wo2'],
         p['Wf1'], p['bf1'].reshape(1, D), p['Wf2'], p['bf2'].reshape(1, D),
         jnp.stack([p['P%d_W1' % k].reshape(2, D, S_DIM)
                    for k in (1, 2, 3, 4)]),
         jnp.stack([p['P%d_b1' % k] for k in (1, 2, 3, 4)]),
         jnp.stack([jnp.pad(p['P%d_W2' % k][:, 0], (0, D - S_DIM))
                    for k in (1, 2, 3, 4)]),
         jnp.stack([p['P%d_b2' % k] for k in (1, 2, 3, 4)]).reshape(1, 4))

    return _tc_dense(ue, ie, iapre, f, xh, nf,
                     ii_sim_lens.astype(jnp.int32),
                     uu_sim_lens.astype(jnp.int32), w)


# speedup vs baseline: 1.5080x; 1.5080x over previous
import functools

import jax
import jax.numpy as jnp
import numpy as np
from jax import lax
from jax.experimental import pallas as pl
from jax.experimental.pallas import tpu as pltpu
from jax.experimental.pallas import tpu_sc as plsc

NUM_USER = 50000
NUM_ITEM = 50000
D = 64
B = 1024
E = 800000
L_HIST = 20
S_SIM = 10
L_SIM = 20
S_DIM = 48

N = NUM_USER + 1
NP = 50176
DUMP = N
RT = NP // 16

EB = 256
SUP = 8
NSUP = 25
NBLK = SUP * NSUP
EPAD = 16 * NBLK * EB
EIDXR = EPAD // EB

DEB = 640
DNBLK = 80


NEG = np.float32(-3.4e38)


def _mesh():
    return plsc.VectorSubcoreMesh(core_axis_name="c", subcore_axis_name="s")


_SC_PARAMS = pltpu.CompilerParams(use_tc_tiling_on_sc=False)


def _sc_degree(dsts2d):
    @functools.partial(
        pl.kernel, mesh=_mesh(),
        out_type=jax.ShapeDtypeStruct((2, NP, 16), jnp.float32),
        scratch_types=[
            pltpu.VMEM((DEB,), jnp.int32),
            pltpu.VMEM((DEB,), jnp.int32),
            pltpu.VMEM((DEB, 16), jnp.float32),
            pltpu.VMEM((DEB, 16), jnp.float32),
            pltpu.VMEM_SHARED((NP, 16), jnp.float32),
            pltpu.SemaphoreType.DMA,
            pltpu.SemaphoreType.DMA,
        ],
        compiler_params=_SC_PARAMS,
    )
    def k(dst_hbm, out_hbm, dbuf0, dbuf1, ones, zbuf, acc, ssem0, ssem1):
        cid = lax.axis_index("c")
        sid = lax.axis_index("s")

        def initrow(i, _):
            ones[i, :] = jnp.full((16,), 1.0, jnp.float32)
            zbuf[i, :] = jnp.full((16,), 0.0, jnp.float32)
            return 0

        lax.fori_loop(0, DEB, initrow, 0)
        r0 = sid * RT
        nz = RT // DEB
        for t in range(nz):
            pltpu.sync_copy(zbuf, acc.at[pl.ds(r0 + t * DEB, DEB)])
        if RT - nz * DEB:
            pltpu.sync_copy(zbuf.at[pl.ds(0, RT - nz * DEB)],
                            acc.at[pl.ds(r0 + nz * DEB, RT - nz * DEB)])
        plsc.subcore_barrier()

        def stage(g, db):
            pltpu.sync_copy(
                dst_hbm.at[cid].at[pl.ds((sid * DNBLK + g) * DEB, DEB)],
                db)

        def fire(db, sem):
            pltpu.async_copy(ones, acc.at[db], sem, add=True)

        def drain(db, sem):
            pltpu.make_async_copy(ones, acc.at[db], sem).wait()

        stage(0, dbuf0)
        fire(dbuf0, ssem0)

        def blk(i2, _):
            g = 2 * i2
            stage(g + 1, dbuf1)
            fire(dbuf1, ssem1)
            drain(dbuf0, ssem0)

            @pl.when(g + 2 < DNBLK)
            def _():
                stage(g + 2, dbuf0)
                fire(dbuf0, ssem0)
            drain(dbuf1, ssem1)
            return 0

        lax.fori_loop(0, DNBLK // 2, blk, 0)
        if DNBLK % 2:
            drain(dbuf0, ssem0)
        plsc.subcore_barrier()
        pltpu.sync_copy(acc.at[pl.ds(r0, RT)],
                        out_hbm.at[cid].at[pl.ds(r0, RT)])

    return k(dsts2d)


def _sc_agg(x2, src256, dst256):
    @functools.partial(
        pl.kernel, mesh=_mesh(),
        out_type=jax.ShapeDtypeStruct((2, NP, 32), jnp.float32),
        scratch_types=[
            pltpu.VMEM((SUP, EB), jnp.int32),
            pltpu.VMEM((SUP, EB), jnp.int32),
            pltpu.VMEM((SUP, EB), jnp.int32),
            pltpu.VMEM((SUP, EB), jnp.int32),
            pltpu.VMEM((EB, 32), jnp.float32),
            pltpu.VMEM((EB, 32), jnp.float32),
            pltpu.VMEM_SHARED((NP, 32), jnp.float32),
            pltpu.SemaphoreType.DMA,
            pltpu.SemaphoreType.DMA,
            pltpu.SemaphoreType.DMA,
            pltpu.SemaphoreType.DMA,
            pltpu.SemaphoreType.DMA,
            pltpu.SemaphoreType.DMA,
        ],
        compiler_params=_SC_PARAMS,
    )
    def k(x2_hbm, src_hbm, dst_hbm, out_hbm, sib0, dib0, sib1, dib1,
          rows0, rows1, acc, isem0, isem1, gsem0, gsem1, ssem0, ssem1):
        cid = lax.axis_index("c")
        sid = lax.axis_index("s")
        sibs = (sib0, sib1)
        dibs = (dib0, dib1)
        isems = (isem0, isem1)
        rowsv = (rows0, rows1)
        gsems = (gsem0, gsem1)
        ssems = (ssem0, ssem1)

        def zrow(i, _):
            rows0[i, 0:16] = jnp.full((16,), 0.0, jnp.float32)
            rows0[i, 16:32] = jnp.full((16,), 0.0, jnp.float32)
            return 0

        lax.fori_loop(0, EB, zrow, 0)
        r0 = sid * RT
        nz = RT // EB
        for t in range(nz):
            pltpu.sync_copy(rows0, acc.at[pl.ds(r0 + t * EB, EB)])
        if RT - nz * EB:
            pltpu.sync_copy(rows0.at[pl.ds(0, RT - nz * EB)],
                            acc.at[pl.ds(r0 + nz * EB, RT - nz * EB)])
        plsc.subcore_barrier()

        def stage_i(sp, w):
            base = (sid * NSUP + sp) * SUP
            pltpu.async_copy(src_hbm.at[pl.ds(base, SUP)], sibs[w], isems[w])
            pltpu.async_copy(dst_hbm.at[pl.ds(base, SUP)], dibs[w], isems[w])

        def drain_i(sp, w):
            base = (sid * NSUP + sp) * SUP
            pltpu.make_async_copy(src_hbm.at[pl.ds(base, SUP)], sibs[w],
                                  isems[w]).wait()
            pltpu.make_async_copy(dst_hbm.at[pl.ds(base, SUP)], dibs[w],
                                  isems[w]).wait()

        def fire_g(w, kk):
            b = kk % 2
            pltpu.async_copy(x2_hbm.at[cid].at[sibs[w].at[kk]], rowsv[b],
                             gsems[b])

        def drain_g(w, kk):
            b = kk % 2
            pltpu.make_async_copy(x2_hbm.at[cid].at[sibs[w].at[kk]],
                                  rowsv[b], gsems[b]).wait()

        def fire_s(w, kk):
            b = kk % 2
            pltpu.async_copy(rowsv[b], acc.at[dibs[w].at[kk]], ssems[b],
                             add=True)

        def drain_s(w, kk):
            b = kk % 2
            pltpu.make_async_copy(rowsv[b], acc.at[dibs[w].at[kk]],
                                  ssems[b]).wait()

        def super_body(sp, w):
            drain_i(sp, w)

            @pl.when(sp + 1 < NSUP)
            def _():
                stage_i(sp + 1, 1 - w)
            fire_g(w, 0)
            for kk in range(SUP):
                if kk >= 1:
                    drain_s(w, kk - 1)
                if kk < SUP - 1:
                    fire_g(w, kk + 1)
                drain_g(w, kk)
                fire_s(w, kk)
            drain_s(w, SUP - 1)

        stage_i(0, 0)

        def body2(i2, _):
            super_body(2 * i2, 0)
            super_body(2 * i2 + 1, 1)
            return 0

        lax.fori_loop(0, NSUP // 2, body2, 0)
        if NSUP % 2:
            super_body(NSUP - 1, 0)
        plsc.subcore_barrier()
        pltpu.sync_copy(acc.at[pl.ds(r0, RT)],
                        out_hbm.at[cid].at[pl.ds(r0, RT)])

    return k(x2, src256, dst256)


def _sc_batch(up, it, user, item, iu2d, iiflat, ui2d, uuflat):
    bs = B // 32

    @functools.partial(
        pl.kernel, mesh=_mesh(),
        out_type=[
            jax.ShapeDtypeStruct((B, 64), jnp.float32),
            jax.ShapeDtypeStruct((B, 64), jnp.float32),
            jax.ShapeDtypeStruct((B, 64), jnp.float32),
            jax.ShapeDtypeStruct((B * S_SIM, 64), jnp.float32),
            jax.ShapeDtypeStruct((B * L_HIST, 64), jnp.float32),
            jax.ShapeDtypeStruct((B * S_SIM, 64), jnp.float32),
        ],
        scratch_types=[
            pltpu.VMEM((bs,), jnp.int32),
            pltpu.VMEM((bs,), jnp.int32),
            pltpu.VMEM((bs * L_HIST // 128, 128), jnp.int32),
            pltpu.VMEM((bs * L_SIM * S_SIM,), jnp.int32),
            pltpu.VMEM((bs, 64), jnp.float32),
            pltpu.VMEM((bs, 64), jnp.float32),
            pltpu.VMEM((bs * L_HIST, 64), jnp.float32),
            pltpu.VMEM((L_SIM * S_SIM, 64), jnp.float32),
            pltpu.VMEM((L_SIM * S_SIM, 64), jnp.float32),
            pltpu.VMEM((bs, 64), jnp.float32),
            pltpu.VMEM((bs * S_SIM, 64), jnp.float32),
            pltpu.VMEM((bs * S_SIM, 64), jnp.float32),
            pltpu.SemaphoreType.DMA,
            pltpu.SemaphoreType.DMA,
            pltpu.SemaphoreType.DMA,
        ],
        compiler_params=_SC_PARAMS,
    )
    def k(up_hbm, it_hbm, u_hbm, i_hbm, iu_hbm, ii_hbm, ui_hbm, uu_hbm,
          ue_out, ie_out, iap_out, f_out, xh_out, nf_out,
          ubuf, ibuf, hbuf, sbuf, uerows, ierows, hrows, srows0, srows1,
          iap, fbuf, nfbuf, sem, sm0, sm1):
        cid = lax.axis_index("c")
        sid = lax.axis_index("s")
        wid = sid * 2 + cid
        gb0 = wid * bs
        hch = bs * L_HIST // 128

        pltpu.sync_copy(u_hbm.at[pl.ds(gb0, bs)], ubuf)
        pltpu.sync_copy(i_hbm.at[pl.ds(gb0, bs)], ibuf)
        pltpu.async_copy(up_hbm.at[ubuf], uerows, sem).wait()
        pltpu.async_copy(it_hbm.at[ibuf], ierows, sem).wait()

        pltpu.sync_copy(iu_hbm.at[pl.ds(wid * hch, hch)], hbuf)
        gd = [pltpu.async_copy(up_hbm.at[hbuf.at[j]],
                               hrows.at[pl.ds(j * 128, 128)], sem)
              for j in range(hch)]
        for dsc in gd:
            dsc.wait()

        def iab(b, _):
            for j in range(4):
                ue16 = uerows[b, pl.ds(j * 16, 16)]

                def lb(l, m):
                    r = hrows[b * L_HIST + l, pl.ds(j * 16, 16)]
                    return jnp.maximum(m, r * ue16)

                m = lax.fori_loop(0, L_HIST, lb,
                                  jnp.full((16,), NEG, jnp.float32))
                iap[b, pl.ds(j * 16, 16)] = m
            return 0

        lax.fori_loop(0, bs, iab, 0)

        pltpu.sync_copy(ui_hbm.at[pl.ds(wid * hch, hch)], hbuf)
        gd = [pltpu.async_copy(it_hbm.at[hbuf.at[j]],
                               hrows.at[pl.ds(j * 128, 128)], sem)
              for j in range(hch)]
        for dsc in gd:
            dsc.wait()
        pltpu.sync_copy(hrows, xh_out.at[pl.ds(gb0 * L_HIST, bs * L_HIST)])

        nsim = L_SIM * S_SIM

        def fire_sim(tab, b, rowbuf, sm):
            pltpu.async_copy(tab.at[sbuf.at[pl.ds(b * nsim, nsim)]],
                             rowbuf, sm)

        def drain_sim(tab, b, rowbuf, sm):
            pltpu.make_async_copy(tab.at[sbuf.at[pl.ds(b * nsim, nsim)]],
                                  rowbuf, sm).wait()

        def pool(b, rowbuf, mrows, obuf):
            def sb(s, _2):
                for j in range(4):
                    m16 = mrows[b, pl.ds(j * 16, 16)]

                    def lb(l, m):
                        r = rowbuf[s * L_SIM + l, pl.ds(j * 16, 16)]
                        return jnp.maximum(m, r * m16)

                    m = lax.fori_loop(0, L_SIM, lb,
                                      jnp.full((16,), NEG, jnp.float32))
                    obuf[b * S_SIM + s, pl.ds(j * 16, 16)] = m
                return 0

            lax.fori_loop(0, S_SIM, sb, 0)

        def sim_pass(tab, mrows, obuf):
            fire_sim(tab, 0, srows0, sm0)

            def b2loop(b2, _):
                b = 2 * b2
                fire_sim(tab, b + 1, srows1, sm1)
                drain_sim(tab, b, srows0, sm0)
                pool(b, srows0, mrows, obuf)

                @pl.when(b + 2 < bs)
                def _():
                    fire_sim(tab, b + 2, srows0, sm0)
                drain_sim(tab, b + 1, srows1, sm1)
                pool(b + 1, srows1, mrows, obuf)
                return 0

            lax.fori_loop(0, bs // 2, b2loop, 0)

        pltpu.sync_copy(ii_hbm.at[pl.ds(gb0 * nsim, bs * nsim)], sbuf)
        sim_pass(up_hbm, uerows, fbuf)
        pltpu.sync_copy(uu_hbm.at[pl.ds(gb0 * nsim, bs * nsim)], sbuf)
        sim_pass(it_hbm, ierows, nfbuf)

        pltpu.sync_copy(uerows, ue_out.at[pl.ds(gb0, bs)])
        pltpu.sync_copy(ierows, ie_out.at[pl.ds(gb0, bs)])
        pltpu.sync_copy(iap, iap_out.at[pl.ds(gb0, bs)])
        pltpu.sync_copy(fbuf, f_out.at[pl.ds(gb0 * S_SIM, bs * S_SIM)])
        pltpu.sync_copy(nfbuf, nf_out.at[pl.ds(gb0 * S_SIM, bs * S_SIM)])

    return k(up, it, user, item, iu2d, iiflat, ui2d, uuflat)


def _tc_prep(tab_pad):
    def body(x_ref, o_ref):
        x = x_ref[...]
        o_ref[0] = x[:, :32]
        o_ref[1] = x[:, 32:]

    return pl.pallas_call(
        body,
        grid=(16,),
        in_specs=[pl.BlockSpec((RT, 64), lambda i: (i, 0))],
        out_specs=pl.BlockSpec((2, RT, 32), lambda i: (0, i, 0)),
        out_shape=jax.ShapeDtypeStruct((2, NP, 32), jnp.float32),
    )(tab_pad)


def _tc_norm(raw, deg16, prev, final):
    def body(raw_ref, deg_ref, prev_ref, *out_refs):
        i = pl.program_id(0)
        raw = raw_ref[...]
        d = jnp.maximum(deg_ref[:, 0:1], 1.0)
        a0 = raw[0] / d
        a1 = raw[1] / d
        nsq = (jnp.sum(a0 * a0, axis=1, keepdims=True)
               + jnp.sum(a1 * a1, axis=1, keepdims=True))
        inv = 1.0 / jnp.maximum(jnp.sqrt(nsq), 1e-12)
        n = jnp.concatenate([a0 * inv, a1 * inv], axis=1)
        if final:
            gi = i * RT + lax.broadcasted_iota(jnp.int32, (RT, 1), 0)
            out = (prev_ref[...] + n) * jnp.float32(1.0 / 3.0)
            out_refs[0][...] = jnp.where(gi == 0, 0.0, out)
        else:
            out_refs[0][0] = a0
            out_refs[0][1] = a1
            out_refs[1][...] = prev_ref[...] + n

    if final:
        out_shape = [jax.ShapeDtypeStruct((NP, 64), jnp.float32)]
        out_specs = [pl.BlockSpec((RT, 64), lambda i: (i, 0))]
    else:
        out_shape = [jax.ShapeDtypeStruct((2, NP, 32), jnp.float32),
                     jax.ShapeDtypeStruct((NP, 64), jnp.float32)]
        out_specs = [pl.BlockSpec((2, RT, 32), lambda i: (0, i, 0)),
                     pl.BlockSpec((RT, 64), lambda i: (i, 0))]
    res = pl.pallas_call(
        body,
        grid=(16,),
        in_specs=[pl.BlockSpec((2, RT, 32), lambda i: (0, i, 0)),
                  pl.BlockSpec((RT, 16), lambda i: (i, 0)),
                  pl.BlockSpec((RT, 64), lambda i: (i, 0))],
        out_specs=out_specs,
        out_shape=out_shape,
    )(raw, deg16, prev)
    return res[0] if final else res


def _tc_dense(ue, ie, iapre, f, xh, nf, lens_ii, lens_uu, w):
    BB = 128
    SQD = np.float32(1.0 / np.sqrt(D))
    SQH = np.float32(1.0 / np.sqrt(32))

    def body(ue_ref, ie_ref, iap_ref, f_ref, xh_ref, nf_ref, li_ref, lu_ref,
             wii_ref, wuu_ref, wq1_ref, wk1_ref, wv1_ref, wo1_ref,
             wq2_ref, wk2_ref, wv2_ref, wo2_ref, wf1_ref, bf1_ref,
             wf2_ref, bf2_ref, w1s_ref, b1s_ref, w2s_ref, b2s_ref, out_ref):
        uev = ue_ref[...]
        iev = ie_ref[...]
        iap = iap_ref[...]
        fv = f_ref[...]
        nfv = nf_ref[...]
        xhv = xh_ref[...]
        li = li_ref[...].reshape(BB)
        lu = lu_ref[...].reshape(BB)

        def sim_fuse(active, fe, lens, wmat):
            act = active @ wmat
            s = jnp.sum(act[:, None, :] * fe, axis=-1) * SQD
            mask = (lax.broadcasted_iota(jnp.int32, (BB, S_SIM), 1)
                    < lens[:, None])
            s = jnp.where(mask, s, -1e9)
            a = jax.nn.softmax(s, axis=-1)
            return jnp.sum(a[:, :, None] * fe, axis=1)

        item_neigh = sim_fuse(iap, fv, li, wii_ref[...])
        ia = 0.5 * (iap + item_neigh)

        xf = xhv.reshape(BB * L_HIST, D)
        q = (xf @ wq1_ref[...]).reshape(BB, L_HIST, D)
        kk = (xf @ wk1_ref[...]).reshape(BB, L_HIST, D)
        vv = (xf @ wv1_ref[...]).reshape(BB, L_HIST, D)
        outs = []
        for h in range(2):
            qh = q[:, :, h * 32:(h + 1) * 32]
            kh = kk[:, :, h * 32:(h + 1) * 32]
            vh = vv[:, :, h * 32:(h + 1) * 32]
            sh = lax.dot_general(qh, kh, (((2,), (2,)), ((0,), (0,))),
                                 preferred_element_type=jnp.float32) * SQH
            ah = jax.nn.softmax(sh, axis=-1)
            oh = lax.dot_general(ah, vh, (((2,), (1,)), ((0,), (0,))),
                                 preferred_element_type=jnp.float32)
            outs.append(oh)
        o = jnp.concatenate(outs, axis=-1).reshape(BB * L_HIST, D)
        x_ = o @ wo1_ref[...]
        h1 = xf + x_
        x = (h1 + jnp.maximum(h1 @ wf1_ref[...] + bf1_ref[...], 0.0)
             @ wf2_ref[...] + bf2_ref[...])
        x3 = x.reshape(BB, L_HIST, D)

        q2 = iev @ wq2_ref[...]
        k2 = (x @ wk2_ref[...]).reshape(BB, L_HIST, D)
        v2 = (x @ wv2_ref[...]).reshape(BB, L_HIST, D)
        s2 = jnp.sum(q2[:, None, :] * k2, axis=-1) * SQD
        a2 = jax.nn.softmax(s2, axis=-1)
        o2 = jnp.sum(a2[:, :, None] * v2, axis=1)
        uiv = o2 @ wo2_ref[...]

        user_neigh = sim_fuse(uiv, nfv, lu, wuu_ref[...])
        ui = 0.5 * (uiv + user_neigh)

        lefts = (uev, ui, uev, ui)
        rights = (iev, iev, ia, ia)
        cols = []
        for kq in range(4):
            hh = (lefts[kq] @ w1s_ref[kq, 0] + rights[kq] @ w1s_ref[kq, 1]
                  + b1s_ref[kq][None, :])
            hh = jnp.maximum(hh, 0.0)
            sc = jnp.sum(hh * w2s_ref[kq, :S_DIM][None, :], axis=1,
                         keepdims=True)
            cols.append(sc)
        out_ref[...] = jnp.concatenate(cols, axis=1) + b2s_ref[...]

    nb = B // BB
    full = lambda shape: pl.BlockSpec(shape, lambda i: tuple(0 for _ in shape))
    in_specs = [
        pl.BlockSpec((BB, 64), lambda i: (i, 0)),
        pl.BlockSpec((BB, 64), lambda i: (i, 0)),
        pl.BlockSpec((BB, 64), lambda i: (i, 0)),
        pl.BlockSpec((BB, S_SIM, 64), lambda i: (i, 0, 0)),
        pl.BlockSpec((BB, L_HIST, 64), lambda i: (i, 0, 0)),
        pl.BlockSpec((BB, S_SIM, 64), lambda i: (i, 0, 0)),
        pl.BlockSpec((1, 1, BB), lambda i: (i, 0, 0)),
        pl.BlockSpec((1, 1, BB), lambda i: (i, 0, 0)),
        full((D, D)), full((D, D)),
        full((D, D)), full((D, D)), full((D, D)), full((D, D)),
        full((D, D)), full((D, D)), full((D, D)), full((D, D)),
        full((D, D)), full((1, D)), full((D, D)), full((1, D)),
        full((4, 2, D, S_DIM)), full((4, S_DIM)), full((4, D)), full((1, 4)),
    ]
    return pl.pallas_call(
        body,
        grid=(nb,),
        in_specs=in_specs,
        out_specs=pl.BlockSpec((BB, 4), lambda i: (i, 0)),
        out_shape=jax.ShapeDtypeStruct((B, 4), jnp.float32),
    )(ue, ie, iapre, f.reshape(B, S_SIM, 64), xh.reshape(B, L_HIST, 64),
      nf.reshape(B, S_SIM, 64),
      lens_ii.reshape(nb, 1, BB), lens_uu.reshape(nb, 1, BB), *w)


def _prep_edges(ei):
    pad = EPAD - E
    spread = N + (jnp.arange(pad, dtype=jnp.int32) % (NP - N))
    src = jnp.concatenate([ei[0].astype(jnp.int32), spread])
    dst = jnp.concatenate([ei[1].astype(jnp.int32), spread])
    return src, dst


def _graph_tables(table, src2d, dst2d, deg16):
    tab_pad = jnp.pad(table, ((0, NP - N), (0, 0)))
    x2 = _tc_prep(tab_pad)
    raw1 = _sc_agg(x2, src2d, dst2d)
    a1, acc1 = _tc_norm(raw1, deg16, tab_pad, final=False)
    raw2 = _sc_agg(a1, src2d, dst2d)
    return _tc_norm(raw2, deg16, acc1, final=True)


def kernel(user, item, user_edge_index, item_edge_index, item_users,
           ii_sim_users, ii_sim_lens, user_items, uu_sim_items, uu_sim_lens,
           params):
    p = params
    usrc, udst = _prep_edges(user_edge_index)
    isrc, idst = _prep_edges(item_edge_index)

    deg2 = _sc_degree(jnp.stack([udst, idst]))
    up_fin = _graph_tables(p['user_table'], usrc.reshape(EIDXR, EB),
                           udst.reshape(EIDXR, EB), deg2[0])
    it_fin = _graph_tables(p['item_table'], isrc.reshape(EIDXR, EB),
                           idst.reshape(EIDXR, EB), deg2[1])

    iu2d = item_users.astype(jnp.int32).reshape(-1, 128)
    ui2d = user_items.astype(jnp.int32).reshape(-1, 128)
    iiflat = ii_sim_users.astype(jnp.int32).reshape(-1)
    uuflat = uu_sim_items.astype(jnp.int32).reshape(-1)

    ue, ie, iapre, f, xh, nf = _sc_batch(
        up_fin, it_fin, user.astype(jnp.int32), item.astype(jnp.int32),
        iu2d, iiflat, ui2d, uuflat)

    w = (p['W_ii'], p['W_uu'],
         p['Wq1'], p['Wk1'], p['Wv1'], p['Wo1'],
         p['Wq2'], p['Wk2'], p['Wv2'], p['Wo2'],
         p['Wf1'], p['bf1'].reshape(1, D), p['Wf2'], p['bf2'].reshape(1, D),
         jnp.stack([p['P%d_W1' % k].reshape(2, D, S_DIM)
                    for k in (1, 2, 3, 4)]),
         jnp.stack([p['P%d_b1' % k] for k in (1, 2, 3, 4)]),
         jnp.stack([jnp.pad(p['P%d_W2' % k][:, 0], (0, D - S_DIM))
                    for k in (1, 2, 3, 4)]),
         jnp.stack([p['P%d_b2' % k] for k in (1, 2, 3, 4)]).reshape(1, 4))

    return _tc_dense(ue, ie, iapre, f, xh, nf,
                     ii_sim_lens.astype(jnp.int32),
                     uu_sim_lens.astype(jnp.int32), w)
